# baseline (speedup 1.0000x reference)
import functools

import jax
import jax.numpy as jnp
from jax import lax
from jax.experimental import pallas as pl
from jax.experimental.pallas import tpu as pltpu
from jax.experimental.pallas import tpu_sc as plsc

B = 4096
D = 32
NC = 2
NS = 16
NW = NC * NS
L = 16
NBUF = 8
NCHUNK = 4
CHUNK = B // NCHUNK


def _make_sc_gather(n_idx):
    per_w = n_idx // NW
    mesh = plsc.VectorSubcoreMesh(core_axis_name="c", subcore_axis_name="s")

    @functools.partial(
        pl.kernel,
        mesh=mesh,
        compiler_params=pltpu.CompilerParams(needs_layout_passes=False),
        out_type=jax.ShapeDtypeStruct((n_idx, D), jnp.float32),
        scratch_types=[
            pltpu.VMEM((per_w,), jnp.int32),
            pltpu.VMEM((per_w, D), jnp.float32),
            pltpu.VMEM((NBUF, D, 128), jnp.float32),
            pltpu.SemaphoreType.DMA,
        ],
    )
    def gather_kernel(emb, idx, rows_out, idx_v, rows, win, sem):
        wid = lax.axis_index("s") * NC + lax.axis_index("c")
        base = wid * per_w
        pltpu.sync_copy(idx.at[pl.ds(base, per_w)], idx_v)
        lanes = lax.iota(jnp.int32, L)
        rlo = lax.iota(jnp.int32, L)
        rhi = rlo + L

        def scalar_at(vec, l):
            return lax.reduce_max(jnp.where(lanes == l, vec, 0), (0,))

        def chunk(j):
            vec = idx_v[pl.ds(j * L, L)]
            for half in range(2):
                copies = []
                for s in range(NBUF):
                    r = scalar_at(vec, half * NBUF + s)
                    w = pl.multiple_of((r >> 7) << 7, 128)
                    copies.append(pltpu.async_copy(
                        emb.at[:, pl.ds(w, 128)], win.at[s], sem))
                for c in copies:
                    c.wait()
                for s in range(NBUF):
                    l = half * NBUF + s
                    i = j * L + l
                    col = lax.broadcast(scalar_at(vec, l) & 127, (L,))
                    rows[i, pl.ds(0, L)] = plsc.load_gather(
                        win.at[s], [rlo, col])
                    rows[i, pl.ds(L, L)] = plsc.load_gather(
                        win.at[s], [rhi, col])

        pl.loop(0, per_w // L)(chunk)
        pltpu.sync_copy(rows, rows_out.at[pl.ds(base, per_w), :])

    return gather_kernel


_gather_full = _make_sc_gather(B)
_gather_chunk = _make_sc_gather(CHUNK)

BM = 512


def _mm_body(w_ref, h_ref, o_ref):
    o_ref[...] = lax.dot_general(
        w_ref[...], h_ref[...],
        (((1,), (1,)), ((), ())),
        preferred_element_type=jnp.float32,
    )


def _mm_body_alias(w_ref, h_ref, _, o_ref):
    _mm_body(w_ref, h_ref, o_ref)


def _mm_chunk(w_chunk, h_i, out_prev, k):
    out_block = pl.BlockSpec(
        (BM, B), lambda i, k=k: (k * (CHUNK // BM) + i, 0))
    if out_prev is None:
        return pl.pallas_call(
            _mm_body,
            grid=(CHUNK // BM,),
            in_specs=[
                pl.BlockSpec((BM, D), lambda i: (i, 0)),
                pl.BlockSpec((B, D), lambda i: (0, 0)),
            ],
            out_specs=out_block,
            out_shape=jax.ShapeDtypeStruct((B, B), jnp.float32),
        )(w_chunk, h_i)
    return pl.pallas_call(
        _mm_body_alias,
        grid=(CHUNK // BM,),
        in_specs=[
            pl.BlockSpec((BM, D), lambda i: (i, 0)),
            pl.BlockSpec((B, D), lambda i: (0, 0)),
            pl.BlockSpec(memory_space=pl.ANY),
        ],
        out_specs=out_block,
        out_shape=jax.ShapeDtypeStruct((B, B), jnp.float32),
        input_output_aliases={2: 0},
    )(w_chunk, h_i, out_prev)


def kernel(user_tensor, item_tensor, user_emb, item_emb):
    h_i = _gather_full(item_emb.T, item_tensor)
    w_chunks = [
        _gather_chunk(user_emb.T, lax.slice(user_tensor, (k * CHUNK,),
                                            ((k + 1) * CHUNK,)))
        for k in range(NCHUNK)
    ]
    out = None
    for k in range(NCHUNK):
        out = _mm_chunk(w_chunks[k], h_i, out, k)
    return out


# speedup vs baseline: 1.1060x; 1.1060x over previous
import functools

import jax
import jax.numpy as jnp
from jax import lax
from jax.experimental import pallas as pl
from jax.experimental.pallas import tpu as pltpu
from jax.experimental.pallas import tpu_sc as plsc

B = 4096
D = 32
NC = 2
NS = 16
NW = NC * NS
L = 16
NBUF = 8
NCHUNK = 2
CHUNK = B // NCHUNK


def _make_sc_gather(n_a, n_b):
    pa, pb = n_a // NW, n_b // NW
    mesh = plsc.VectorSubcoreMesh(core_axis_name="c", subcore_axis_name="s")
    out_type = [jax.ShapeDtypeStruct((n_a, D), jnp.float32)]
    scratch = [
        pltpu.VMEM((pa,), jnp.int32),
        pltpu.VMEM((pa, D), jnp.float32),
        pltpu.VMEM((NBUF, D, 128), jnp.float32),
        pltpu.SemaphoreType.DMA,
    ]
    if n_b:
        out_type.append(jax.ShapeDtypeStruct((n_b, D), jnp.float32))
        scratch += [
            pltpu.VMEM((pb,), jnp.int32),
            pltpu.VMEM((pb, D), jnp.float32),
            pltpu.VMEM((NBUF, D, 128), jnp.float32),
            pltpu.SemaphoreType.DMA,
        ]

    @functools.partial(
        pl.kernel,
        mesh=mesh,
        compiler_params=pltpu.CompilerParams(needs_layout_passes=False),
        out_type=tuple(out_type) if n_b else out_type[0],
        scratch_types=scratch,
    )
    def gather_kernel(*refs):
        if n_b:
            (emb_a, emb_b, idx_a, idx_b, out_a, out_b,
             idx_va, rows_a, win_a, sem_a,
             idx_vb, rows_b, win_b, sem_b) = refs
            tabs = [(emb_a, idx_a, out_a, idx_va, rows_a, win_a, sem_a, pa),
                    (emb_b, idx_b, out_b, idx_vb, rows_b, win_b, sem_b, pb)]
        else:
            emb_a, idx_a, out_a, idx_va, rows_a, win_a, sem_a = refs
            tabs = [(emb_a, idx_a, out_a, idx_va, rows_a, win_a, sem_a, pa)]
        wid = lax.axis_index("s") * NC + lax.axis_index("c")
        lanes = lax.iota(jnp.int32, L)
        rlo = lax.iota(jnp.int32, L)
        rhi = rlo + L

        def scalar_at(vec, l):
            return lax.reduce_max(jnp.where(lanes == l, vec, 0), (0,))

        for emb, idx, _, idx_v, _, _, _, p in tabs:
            pltpu.sync_copy(idx.at[pl.ds(wid * p, p)], idx_v)

        for emb, idx, out, idx_v, rows, win, sem, p in tabs:
            def chunk(j, emb=emb, idx_v=idx_v, rows=rows, win=win, sem=sem):
                vec = idx_v[pl.ds(j * L, L)]
                for half in range(2):
                    copies = []
                    for s in range(NBUF):
                        r = scalar_at(vec, half * NBUF + s)
                        w = pl.multiple_of((r >> 7) << 7, 128)
                        copies.append(pltpu.async_copy(
                            emb.at[:, pl.ds(w, 128)], win.at[s], sem))
                    for c in copies:
                        c.wait()
                    for s in range(NBUF):
                        l = half * NBUF + s
                        i = j * L + l
                        col = lax.broadcast(scalar_at(vec, l) & 127, (L,))
                        rows[i, pl.ds(0, L)] = plsc.load_gather(
                            win.at[s], [rlo, col])
                        rows[i, pl.ds(L, L)] = plsc.load_gather(
                            win.at[s], [rhi, col])

            pl.loop(0, p // L)(chunk)
        for _, _, out, _, rows, _, _, p in tabs:
            pltpu.sync_copy(rows, out.at[pl.ds(wid * p, p), :])

    return gather_kernel

BM = 512


def _mm_body(w_ref, h_ref, o_ref):
    o_ref[...] = lax.dot_general(
        w_ref[...], h_ref[...],
        (((1,), (1,)), ((), ())),
        preferred_element_type=jnp.float32,
    )


def _mm_body_alias(w_ref, h_ref, _, o_ref):
    _mm_body(w_ref, h_ref, o_ref)


def _mm_chunk(w_chunk, h_i, out_prev, k):
    out_block = pl.BlockSpec(
        (BM, B), lambda i, k=k: (k * (CHUNK // BM) + i, 0))
    if out_prev is None:
        return pl.pallas_call(
            _mm_body,
            grid=(CHUNK // BM,),
            in_specs=[
                pl.BlockSpec((BM, D), lambda i: (i, 0)),
                pl.BlockSpec((B, D), lambda i: (0, 0)),
            ],
            out_specs=out_block,
            out_shape=jax.ShapeDtypeStruct((B, B), jnp.float32),
        )(w_chunk, h_i)
    return pl.pallas_call(
        _mm_body_alias,
        grid=(CHUNK // BM,),
        in_specs=[
            pl.BlockSpec((BM, D), lambda i: (i, 0)),
            pl.BlockSpec((B, D), lambda i: (0, 0)),
            pl.BlockSpec(memory_space=pl.ANY),
        ],
        out_specs=out_block,
        out_shape=jax.ShapeDtypeStruct((B, B), jnp.float32),
        input_output_aliases={2: 0},
    )(w_chunk, h_i, out_prev)


_gather_ab = _make_sc_gather(B, CHUNK)
_gather_b = _make_sc_gather(CHUNK, 0)


def kernel(user_tensor, item_tensor, user_emb, item_emb):
    u0 = lax.slice(user_tensor, (0,), (CHUNK,))
    u1 = lax.slice(user_tensor, (CHUNK,), (B,))
    h_i, w0 = _gather_ab(item_emb.T, user_emb.T, item_tensor, u0)
    w1 = _gather_b(user_emb.T, u1)
    out = _mm_chunk(w0, h_i, None, 0)
    out = _mm_chunk(w1, h_i, out, 1)
    return out
